# Initial kernel scaffold; baseline (speedup 1.0000x reference)
import jax
import jax.numpy as jnp
from jax.experimental import pallas as pl


def kernel(x, edge_index, batch, edge_weight, W1, b1, W2, b2, Wfc, bfc):
    raise NotImplementedError("write your pallas kernel here")


# speedup vs baseline: 19.3787x; 19.3787x over previous
import functools

import jax
import jax.numpy as jnp
from jax import lax
from jax.experimental import pallas as pl
from jax.experimental.pallas import tpu as pltpu
from jax.experimental.pallas import tpu_sc as plsc

N = 10000
E = 320000
F_IN = 128
H1 = 8
H2 = 16
G = 16

NTILES = 16
NP = 10240
RPT = NP // NTILES
C = 128
EPT = E // NTILES
NCH = (EPT + C - 1) // C
EPTP = NCH * C
EP = EPTP * NTILES


def _scf32(shape):
    return pltpu.VMEM(shape, jnp.float32)


def _sc_body(rowp, colp, ewp, y1p, b1p, h4,
             row2, col2, ew2, norm2, dis_l, comb, disc,
             gath, msg, rb, zb, b1l,
             deg_sh, dis_sh, hA, hB):
    cid = lax.axis_index("c")
    wid = lax.axis_index("s")

    iot = lax.iota(jnp.int32, 16)
    hi = lax.shift_right_logical(iot, 3)
    lo = lax.bitwise_and(iot, 7)
    zf = jnp.zeros((16,), jnp.float32)

    @pl.when(cid == 0)
    def _main():
        r0 = wid * RPT

        pltpu.sync_copy(rowp.at[wid], row2)
        pltpu.sync_copy(colp.at[wid], col2)
        pltpu.sync_copy(ewp.at[wid], ew2)
        pltpu.sync_copy(b1p, b1l)

        for v in range(C * H1 // 16):
            plsc.store_scatter(zb, [hi + (2 * v), lo], zf)
            plsc.store_scatter(msg, [hi + (2 * v), lo], zf)
        for s in range(RPT // C):
            pltpu.sync_copy(zb, deg_sh.at[pl.ds(r0 + s * C, C)])
        plsc.subcore_barrier()

        zcol = jnp.zeros((16,), jnp.int32)
        def _deg(j, _):
            for v in range(C // 16):
                w = ew2[j, pl.ds(v * 16, 16)]
                plsc.store_scatter(msg, [iot + 16 * v, zcol], w)
            pltpu.sync_copy(msg, deg_sh.at[col2.at[j]], add=True)
            return 0
        lax.fori_loop(0, NCH, _deg, 0)
        plsc.subcore_barrier()

        for s in range(RPT // C):
            base = r0 + s * C
            pltpu.sync_copy(deg_sh.at[pl.ds(base, C)], comb)
            for v in range(C // 16):
                acc = plsc.load_gather(comb, [iot + 16 * v, zcol])
                m = acc > 0.0
                xs = jnp.where(m, acc, 1.0)
                i32 = plsc.bitcast(xs, jnp.int32)
                i32 = 1597463007 - lax.shift_right_arithmetic(i32, 1)
                y = plsc.bitcast(i32, jnp.float32)
                for _ in range(3):
                    y = y * (1.5 - 0.5 * xs * y * y)
                disc[pl.ds(v * 16, 16)] = jnp.where(m, y, 0.0)
            pltpu.sync_copy(disc, dis_sh.at[pl.ds(base, C)])
        plsc.subcore_barrier()
        pltpu.sync_copy(dis_sh, dis_l)

        def _norm(j, _):
            for v in range(C // 16):
                r = row2[j, pl.ds(v * 16, 16)]
                c = col2[j, pl.ds(v * 16, 16)]
                w = ew2[j, pl.ds(v * 16, 16)]
                n = plsc.load_gather(dis_l, [r]) * w * plsc.load_gather(dis_l, [c])
                norm2[j, pl.ds(v * 16, 16)] = n
            return 0
        lax.fori_loop(0, NCH, _norm, 0)
        plsc.subcore_barrier()

        def prop(src_sh, dst_sh):
            def _p(j, _):
                pltpu.sync_copy(src_sh.at[row2.at[j]], gath)
                for v in range(C // 2):
                    i0 = hi + (2 * v)
                    nrm = plsc.load_gather(norm2.at[j], [i0])
                    g = plsc.load_gather(gath, [i0, lo])
                    plsc.store_scatter(msg, [i0, lo], g * nrm)
                pltpu.sync_copy(msg, dst_sh.at[col2.at[j]], add=True)
                return 0
            lax.fori_loop(0, NCH, _p, 0)
            plsc.subcore_barrier()

        def init_cols(dst_sh, k):
            pltpu.sync_copy(y1p.at[k, pl.ds(r0, RPT)],
                            dst_sh.at[pl.ds(r0, RPT)])

        def zero_sh(dst_sh):
            for s in range(RPT // C):
                pltpu.sync_copy(zb, dst_sh.at[pl.ds(r0 + s * C, C)])

        init_cols(hA, 3)
        init_cols(hB, 2)
        plsc.subcore_barrier()
        prop(hA, hB)
        init_cols(hA, 1)
        plsc.subcore_barrier()
        prop(hB, hA)
        init_cols(hB, 0)
        plsc.subcore_barrier()
        prop(hA, hB)

        pltpu.sync_copy(hB.at[pl.ds(r0, RPT)], rb)
        bvec = b1l[pl.ds(0, 16)]
        def _relu(v, _):
            i0 = hi + (2 * v)
            val = plsc.load_gather(rb, [i0, lo])
            val = jnp.maximum(val + bvec, 0.0)
            plsc.store_scatter(rb, [i0, lo], val)
            return 0
        lax.fori_loop(0, RPT * H1 // 16, _relu, 0)
        pltpu.sync_copy(rb, hA.at[pl.ds(r0, RPT)])
        pltpu.sync_copy(rb, h4.at[0, pl.ds(r0, RPT)])
        plsc.subcore_barrier()

        zero_sh(hB)
        plsc.subcore_barrier()
        prop(hA, hB)
        pltpu.sync_copy(hB.at[pl.ds(r0, RPT)], h4.at[1, pl.ds(r0, RPT)])
        zero_sh(hA)
        plsc.subcore_barrier()
        prop(hB, hA)
        pltpu.sync_copy(hA.at[pl.ds(r0, RPT)], h4.at[2, pl.ds(r0, RPT)])
        zero_sh(hB)
        plsc.subcore_barrier()
        prop(hA, hB)
        pltpu.sync_copy(hB.at[pl.ds(r0, RPT)], h4.at[3, pl.ds(r0, RPT)])


def _make_sc_kernel():
    mesh = plsc.VectorSubcoreMesh(core_axis_name="c", subcore_axis_name="s")
    scratch = [
        pltpu.VMEM((NCH, C), jnp.int32),
        pltpu.VMEM((NCH, C), jnp.int32),
        _scf32((NCH, C)),
        _scf32((NCH, C)),
        _scf32((NP,)),
        _scf32((C, H1)),
        _scf32((C,)),
        _scf32((C, H1)),
        _scf32((C, H1)),
        _scf32((RPT, H1)),
        _scf32((C, H1)),
        _scf32((16,)),
        pltpu.VMEM_SHARED((NP, H1), jnp.float32),
        pltpu.VMEM_SHARED((NP,), jnp.float32),
        pltpu.VMEM_SHARED((NP, H1), jnp.float32),
        pltpu.VMEM_SHARED((NP, H1), jnp.float32),
    ]
    return pl.kernel(
        _sc_body,
        out_type=jax.ShapeDtypeStruct((4, NP, H1), jnp.float32),
        mesh=mesh,
        scratch_types=scratch,
        compiler_params=pltpu.CompilerParams(
            needs_layout_passes=False, use_tc_tiling_on_sc=False),
    )


def _tc_proj(x_ref, w_ref, o_ref):
    o_ref[:] = jnp.dot(x_ref[:], w_ref[:], preferred_element_type=jnp.float32)


def _tc_head(h4_ref, w2_ref, b2_ref, batch_ref, batchc_ref, wfc_ref, bfc_ref, o_ref):
    h4 = h4_ref[:]
    h2 = jnp.maximum(jnp.dot(h4, w2_ref[:], preferred_element_type=jnp.float32)
                     + b2_ref[:], 0.0)
    batch = batch_ref[:]
    gids = lax.broadcasted_iota(jnp.int32, (G, NP), 0)
    oh = (gids == batch).astype(jnp.float32)
    cnt = jnp.sum(oh, axis=1, keepdims=True)
    sums = jnp.dot(oh, h2, preferred_element_type=jnp.float32)
    mean = sums / jnp.clip(cnt, 1.0)
    neg = jnp.float32(-3.0e38)
    batchc = batchc_ref[:]
    mxs = []
    for g in range(G):
        maskg = batchc == g
        mxs.append(jnp.max(jnp.where(maskg, h2, neg), axis=0, keepdims=True))
    mx = jnp.concatenate(mxs, axis=0)
    mx = jnp.where(cnt > 0.0, mx, 0.0)
    pooled = jnp.concatenate([mx, mean], axis=1)
    o_ref[:] = jnp.dot(pooled, wfc_ref[:], preferred_element_type=jnp.float32) + bfc_ref[:]


@jax.jit
def kernel(x, edge_index, batch, edge_weight, W1, b1, W2, b2, Wfc, bfc):
    W1s = jnp.transpose(W1, (1, 0, 2)).reshape(F_IN, 32)
    W2s = W2.reshape(32, H2)

    row = edge_index[0]
    col = edge_index[1]
    pad_e = EP - E
    rowp = jnp.pad(row, (0, pad_e)).reshape(NTILES, NCH, C)
    colp = jnp.pad(col, (0, pad_e)).reshape(NTILES, NCH, C)
    ewp = jnp.pad(edge_weight, (0, pad_e)).reshape(NTILES, NCH, C)
    b1p = jnp.concatenate([b1, b1])

    y1 = pl.pallas_call(
        _tc_proj,
        out_shape=jax.ShapeDtypeStruct((N, 32), jnp.float32),
    )(x, W1s)
    y1p = jnp.pad(y1, ((0, NP - N), (0, 0)))
    y1p = jnp.transpose(y1p.reshape(NP, 4, H1), (1, 0, 2))

    h4 = _make_sc_kernel()(rowp, colp, ewp, y1p, b1p)
    h4 = jnp.transpose(h4, (1, 0, 2)).reshape(NP, 32)

    batchp = jnp.pad(batch, (0, NP - N), constant_values=G + 7)
    out = pl.pallas_call(
        _tc_head,
        out_shape=jax.ShapeDtypeStruct((G, 2), jnp.float32),
    )(h4, W2s, b2.reshape(1, H2), batchp.reshape(1, NP),
      batchp.reshape(NP, 1), Wfc, bfc.reshape(1, 2))
    return out


# speedup vs baseline: 21.5059x; 1.1098x over previous
import functools

import jax
import jax.numpy as jnp
from jax import lax
from jax.experimental import pallas as pl
from jax.experimental.pallas import tpu as pltpu
from jax.experimental.pallas import tpu_sc as plsc

N = 10000
E = 320000
F_IN = 128
H1 = 8
H2 = 16
G = 16

NTILES = 16
NP = 10240
RPT = NP // NTILES
C = 128
EPT = E // NTILES
NCH = (EPT + C - 1) // C
EPTP = NCH * C
EP = EPTP * NTILES


def _scf32(shape):
    return pltpu.VMEM(shape, jnp.float32)


def _sc_body(rowp, colp, ewp, y1p, b1p, h4,
             row2, col2, ew2, norm2, dis_l, comb, disc,
             gath, msg, gath2, msg2, gsem, gsem2, rb, zb, b1l,
             deg_sh, dis_sh, hA, hB):
    cid = lax.axis_index("c")
    wid = lax.axis_index("s")

    iot = lax.iota(jnp.int32, 16)
    hi = lax.shift_right_logical(iot, 3)
    lo = lax.bitwise_and(iot, 7)
    zf = jnp.zeros((16,), jnp.float32)

    @pl.when(cid == 0)
    def _main():
        r0 = wid * RPT

        pltpu.sync_copy(rowp.at[wid], row2)
        pltpu.sync_copy(colp.at[wid], col2)
        pltpu.sync_copy(ewp.at[wid], ew2)
        pltpu.sync_copy(b1p, b1l)

        for v in range(C * H1 // 16):
            plsc.store_scatter(zb, [hi + (2 * v), lo], zf)
            plsc.store_scatter(msg, [hi + (2 * v), lo], zf)
        for s in range(RPT // C):
            pltpu.sync_copy(zb, deg_sh.at[pl.ds(r0 + s * C, C)])
        plsc.subcore_barrier()

        zcol = jnp.zeros((16,), jnp.int32)
        def _deg(j, _):
            for v in range(C // 16):
                w = ew2[j, pl.ds(v * 16, 16)]
                plsc.store_scatter(msg, [iot + 16 * v, zcol], w)
            pltpu.sync_copy(msg, deg_sh.at[col2.at[j]], add=True)
            return 0
        lax.fori_loop(0, NCH, _deg, 0)
        plsc.subcore_barrier()

        for s in range(RPT // C):
            base = r0 + s * C
            pltpu.sync_copy(deg_sh.at[pl.ds(base, C)], comb)
            for v in range(C // 16):
                acc = plsc.load_gather(comb, [iot + 16 * v, zcol])
                m = acc > 0.0
                xs = jnp.where(m, acc, 1.0)
                i32 = plsc.bitcast(xs, jnp.int32)
                i32 = 1597463007 - lax.shift_right_arithmetic(i32, 1)
                y = plsc.bitcast(i32, jnp.float32)
                for _ in range(3):
                    y = y * (1.5 - 0.5 * xs * y * y)
                disc[pl.ds(v * 16, 16)] = jnp.where(m, y, 0.0)
            pltpu.sync_copy(disc, dis_sh.at[pl.ds(base, C)])
        plsc.subcore_barrier()
        pltpu.sync_copy(dis_sh, dis_l)

        def _norm(j, _):
            for v in range(C // 16):
                r = row2[j, pl.ds(v * 16, 16)]
                c = col2[j, pl.ds(v * 16, 16)]
                w = ew2[j, pl.ds(v * 16, 16)]
                n = plsc.load_gather(dis_l, [r]) * w * plsc.load_gather(dis_l, [c])
                norm2[j, pl.ds(v * 16, 16)] = n
            return 0
        lax.fori_loop(0, NCH, _norm, 0)
        plsc.subcore_barrier()

        def prop(src_sh, dst_sh):
            def scale(gbuf, mbuf, j):
                for v in range(C // 2):
                    i0 = hi + (2 * v)
                    nrm = plsc.load_gather(norm2.at[j], [i0])
                    g = plsc.load_gather(gbuf, [i0, lo])
                    plsc.store_scatter(mbuf, [i0, lo], g * nrm)

            pltpu.async_copy(src_sh.at[row2.at[0]], gath, gsem)

            def _p(t, _):
                j0 = 2 * t
                j1 = j0 + 1
                pltpu.make_async_copy(src_sh.at[row2.at[j0]], gath, gsem).wait()
                pltpu.async_copy(src_sh.at[row2.at[j1]], gath2, gsem2)
                scale(gath, msg, j0)
                pltpu.sync_copy(msg, dst_sh.at[col2.at[j0]], add=True)
                pltpu.make_async_copy(src_sh.at[row2.at[j1]], gath2, gsem2).wait()

                @pl.when(t < (NCH - 1) // 2 - 1)
                def _pref():
                    pltpu.async_copy(src_sh.at[row2.at[j0 + 2]], gath, gsem)
                scale(gath2, msg2, j1)
                pltpu.sync_copy(msg2, dst_sh.at[col2.at[j1]], add=True)
                return 0
            lax.fori_loop(0, (NCH - 1) // 2, _p, 0)
            jl = NCH - 1
            pltpu.sync_copy(src_sh.at[row2.at[jl]], gath)
            scale(gath, msg, jl)
            pltpu.sync_copy(msg, dst_sh.at[col2.at[jl]], add=True)
            plsc.subcore_barrier()

        def init_cols(dst_sh, k):
            pltpu.sync_copy(y1p.at[k, pl.ds(r0, RPT)],
                            dst_sh.at[pl.ds(r0, RPT)])

        def zero_sh(dst_sh):
            for s in range(RPT // C):
                pltpu.sync_copy(zb, dst_sh.at[pl.ds(r0 + s * C, C)])

        init_cols(hA, 3)
        init_cols(hB, 2)
        plsc.subcore_barrier()
        prop(hA, hB)
        init_cols(hA, 1)
        plsc.subcore_barrier()
        prop(hB, hA)
        init_cols(hB, 0)
        plsc.subcore_barrier()
        prop(hA, hB)

        pltpu.sync_copy(hB.at[pl.ds(r0, RPT)], rb)
        bvec = b1l[pl.ds(0, 16)]
        def _relu(v, _):
            i0 = hi + (2 * v)
            val = plsc.load_gather(rb, [i0, lo])
            val = jnp.maximum(val + bvec, 0.0)
            plsc.store_scatter(rb, [i0, lo], val)
            return 0
        lax.fori_loop(0, RPT * H1 // 16, _relu, 0)
        pltpu.sync_copy(rb, hA.at[pl.ds(r0, RPT)])
        pltpu.sync_copy(rb, h4.at[0, pl.ds(r0, RPT)])
        plsc.subcore_barrier()

        zero_sh(hB)
        plsc.subcore_barrier()
        prop(hA, hB)
        pltpu.sync_copy(hB.at[pl.ds(r0, RPT)], h4.at[1, pl.ds(r0, RPT)])
        zero_sh(hA)
        plsc.subcore_barrier()
        prop(hB, hA)
        pltpu.sync_copy(hA.at[pl.ds(r0, RPT)], h4.at[2, pl.ds(r0, RPT)])
        zero_sh(hB)
        plsc.subcore_barrier()
        prop(hA, hB)
        pltpu.sync_copy(hB.at[pl.ds(r0, RPT)], h4.at[3, pl.ds(r0, RPT)])


def _make_sc_kernel():
    mesh = plsc.VectorSubcoreMesh(core_axis_name="c", subcore_axis_name="s")
    scratch = [
        pltpu.VMEM((NCH, C), jnp.int32),
        pltpu.VMEM((NCH, C), jnp.int32),
        _scf32((NCH, C)),
        _scf32((NCH, C)),
        _scf32((NP,)),
        _scf32((C, H1)),
        _scf32((C,)),
        _scf32((C, H1)),
        _scf32((C, H1)),
        _scf32((C, H1)),
        _scf32((C, H1)),
        pltpu.SemaphoreType.DMA,
        pltpu.SemaphoreType.DMA,
        _scf32((RPT, H1)),
        _scf32((C, H1)),
        _scf32((16,)),
        pltpu.VMEM_SHARED((NP, H1), jnp.float32),
        pltpu.VMEM_SHARED((NP,), jnp.float32),
        pltpu.VMEM_SHARED((NP, H1), jnp.float32),
        pltpu.VMEM_SHARED((NP, H1), jnp.float32),
    ]
    return pl.kernel(
        _sc_body,
        out_type=jax.ShapeDtypeStruct((4, NP, H1), jnp.float32),
        mesh=mesh,
        scratch_types=scratch,
        compiler_params=pltpu.CompilerParams(
            needs_layout_passes=False, use_tc_tiling_on_sc=False),
    )


def _tc_proj(x_ref, w_ref, o_ref):
    o_ref[:] = jnp.dot(x_ref[:], w_ref[:], preferred_element_type=jnp.float32)


def _tc_head(h4_ref, w2_ref, b2_ref, batch_ref, batchc_ref, wfc_ref, bfc_ref, o_ref):
    h4 = h4_ref[:]
    h2 = jnp.maximum(jnp.dot(h4, w2_ref[:], preferred_element_type=jnp.float32)
                     + b2_ref[:], 0.0)
    batch = batch_ref[:]
    gids = lax.broadcasted_iota(jnp.int32, (G, NP), 0)
    oh = (gids == batch).astype(jnp.float32)
    cnt = jnp.sum(oh, axis=1, keepdims=True)
    sums = jnp.dot(oh, h2, preferred_element_type=jnp.float32)
    mean = sums / jnp.clip(cnt, 1.0)
    neg = jnp.float32(-3.0e38)
    batchc = batchc_ref[:]
    mxs = []
    for g in range(G):
        maskg = batchc == g
        mxs.append(jnp.max(jnp.where(maskg, h2, neg), axis=0, keepdims=True))
    mx = jnp.concatenate(mxs, axis=0)
    mx = jnp.where(cnt > 0.0, mx, 0.0)
    pooled = jnp.concatenate([mx, mean], axis=1)
    o_ref[:] = jnp.dot(pooled, wfc_ref[:], preferred_element_type=jnp.float32) + bfc_ref[:]


@jax.jit
def kernel(x, edge_index, batch, edge_weight, W1, b1, W2, b2, Wfc, bfc):
    W1s = jnp.transpose(W1, (1, 0, 2)).reshape(F_IN, 32)
    W2s = W2.reshape(32, H2)

    row = edge_index[0]
    col = edge_index[1]
    pad_e = EP - E
    rowp = jnp.pad(row, (0, pad_e)).reshape(NTILES, NCH, C)
    colp = jnp.pad(col, (0, pad_e)).reshape(NTILES, NCH, C)
    ewp = jnp.pad(edge_weight, (0, pad_e)).reshape(NTILES, NCH, C)
    b1p = jnp.concatenate([b1, b1])

    y1 = pl.pallas_call(
        _tc_proj,
        out_shape=jax.ShapeDtypeStruct((N, 32), jnp.float32),
    )(x, W1s)
    y1p = jnp.pad(y1, ((0, NP - N), (0, 0)))
    y1p = jnp.transpose(y1p.reshape(NP, 4, H1), (1, 0, 2))

    h4 = _make_sc_kernel()(rowp, colp, ewp, y1p, b1p)
    h4 = jnp.transpose(h4, (1, 0, 2)).reshape(NP, 32)

    batchp = jnp.pad(batch, (0, NP - N), constant_values=G + 7)
    out = pl.pallas_call(
        _tc_head,
        out_shape=jax.ShapeDtypeStruct((G, 2), jnp.float32),
    )(h4, W2s, b2.reshape(1, H2), batchp.reshape(1, NP),
      batchp.reshape(NP, 1), Wfc, bfc.reshape(1, 2))
    return out


# speedup vs baseline: 25.3504x; 1.1788x over previous
import functools

import jax
import jax.numpy as jnp
from jax import lax
from jax.experimental import pallas as pl
from jax.experimental.pallas import tpu as pltpu
from jax.experimental.pallas import tpu_sc as plsc

N = 10000
E = 320000
F_IN = 128
H1 = 8
H2 = 16
G = 16

NTILES = 16
NP = 10240
RPT = NP // NTILES
C = 128
EPT = E // NTILES
NCH = (EPT + C - 1) // C
EPTP = NCH * C
EP = EPTP * NTILES


def _scf32(shape):
    return pltpu.VMEM(shape, jnp.float32)


def _sc_body(rowp, colp, ewp, y1p, b1p, h4,
             row2, col2, ew2, norm2, dis_l, comb, disc,
             gath, msg, gath2, msg2, gsem, gsem2, ssem, rb, zb, b1l,
             deg_sh, dis_sh, hA, hB):
    cid = lax.axis_index("c")
    wid = lax.axis_index("s")

    iot = lax.iota(jnp.int32, 16)
    hi = lax.shift_right_logical(iot, 3)
    lo = lax.bitwise_and(iot, 7)
    zf = jnp.zeros((16,), jnp.float32)

    @pl.when(cid == 0)
    def _main():
        r0 = wid * RPT

        pltpu.sync_copy(rowp.at[wid], row2)
        pltpu.sync_copy(colp.at[wid], col2)
        pltpu.sync_copy(ewp.at[wid], ew2)
        pltpu.sync_copy(b1p, b1l)

        for v in range(C * H1 // 16):
            plsc.store_scatter(zb, [hi + (2 * v), lo], zf)
            plsc.store_scatter(msg, [hi + (2 * v), lo], zf)
        for s in range(RPT // C):
            pltpu.sync_copy(zb, deg_sh.at[pl.ds(r0 + s * C, C)])
        plsc.subcore_barrier()

        zcol = jnp.zeros((16,), jnp.int32)
        def _deg(j, _):
            for v in range(C // 16):
                w = ew2[j, pl.ds(v * 16, 16)]
                plsc.store_scatter(msg, [iot + 16 * v, zcol], w)
            pltpu.sync_copy(msg, deg_sh.at[col2.at[j]], add=True)
            return 0
        lax.fori_loop(0, NCH, _deg, 0)
        plsc.subcore_barrier()

        for s in range(RPT // C):
            base = r0 + s * C
            pltpu.sync_copy(deg_sh.at[pl.ds(base, C)], comb)
            for v in range(C // 16):
                acc = plsc.load_gather(comb, [iot + 16 * v, zcol])
                m = acc > 0.0
                xs = jnp.where(m, acc, 1.0)
                i32 = plsc.bitcast(xs, jnp.int32)
                i32 = 1597463007 - lax.shift_right_arithmetic(i32, 1)
                y = plsc.bitcast(i32, jnp.float32)
                for _ in range(3):
                    y = y * (1.5 - 0.5 * xs * y * y)
                disc[pl.ds(v * 16, 16)] = jnp.where(m, y, 0.0)
            pltpu.sync_copy(disc, dis_sh.at[pl.ds(base, C)])
        plsc.subcore_barrier()
        pltpu.sync_copy(dis_sh, dis_l)

        def _norm(j, _):
            for v in range(C // 16):
                r = row2[j, pl.ds(v * 16, 16)]
                c = col2[j, pl.ds(v * 16, 16)]
                w = ew2[j, pl.ds(v * 16, 16)]
                n = plsc.load_gather(dis_l, [r]) * w * plsc.load_gather(dis_l, [c])
                norm2[j, pl.ds(v * 16, 16)] = n
            return 0
        lax.fori_loop(0, NCH, _norm, 0)
        plsc.subcore_barrier()

        def prop(src_sh, dst_sh):
            def scale(gbuf, mbuf, j):
                for w in range(C // 16):
                    nch = norm2[j, pl.ds(w * 16, 16)]
                    for u in range(8):
                        v = w * 8 + u
                        i0 = hi + (2 * v)
                        nrm = nch.at[hi + (2 * u)].get(mode="promise_in_bounds")
                        g = plsc.load_gather(gbuf, [i0, lo])
                        plsc.store_scatter(mbuf, [i0, lo], g * nrm)

            pltpu.async_copy(src_sh.at[row2.at[0]], gath, gsem)

            def _p(t, _):
                j0 = 2 * t
                j1 = j0 + 1
                pltpu.make_async_copy(src_sh.at[row2.at[j0]], gath, gsem).wait()
                pltpu.async_copy(src_sh.at[row2.at[j1]], gath2, gsem2)
                scale(gath, msg, j0)
                d0 = pltpu.make_async_copy(msg, dst_sh.at[col2.at[j0]], ssem)
                d0.start(add=True)
                pltpu.make_async_copy(src_sh.at[row2.at[j1]], gath2, gsem2).wait()

                @pl.when(t < (NCH - 1) // 2 - 1)
                def _pref():
                    pltpu.async_copy(src_sh.at[row2.at[j0 + 2]], gath, gsem)
                scale(gath2, msg2, j1)
                d0.wait()
                pltpu.sync_copy(msg2, dst_sh.at[col2.at[j1]], add=True)
                return 0
            lax.fori_loop(0, (NCH - 1) // 2, _p, 0)
            jl = NCH - 1
            pltpu.sync_copy(src_sh.at[row2.at[jl]], gath)
            scale(gath, msg, jl)
            pltpu.sync_copy(msg, dst_sh.at[col2.at[jl]], add=True)
            plsc.subcore_barrier()

        def init_cols(dst_sh, k):
            pltpu.sync_copy(y1p.at[k, pl.ds(r0, RPT)],
                            dst_sh.at[pl.ds(r0, RPT)])

        def zero_sh(dst_sh):
            for s in range(RPT // C):
                pltpu.sync_copy(zb, dst_sh.at[pl.ds(r0 + s * C, C)])

        init_cols(hA, 3)
        init_cols(hB, 2)
        plsc.subcore_barrier()
        prop(hA, hB)
        init_cols(hA, 1)
        plsc.subcore_barrier()
        prop(hB, hA)
        init_cols(hB, 0)
        plsc.subcore_barrier()
        prop(hA, hB)

        pltpu.sync_copy(hB.at[pl.ds(r0, RPT)], rb)
        bvec = b1l[pl.ds(0, 16)]
        def _relu(v, _):
            i0 = hi + (2 * v)
            val = plsc.load_gather(rb, [i0, lo])
            val = jnp.maximum(val + bvec, 0.0)
            plsc.store_scatter(rb, [i0, lo], val)
            return 0
        lax.fori_loop(0, RPT * H1 // 16, _relu, 0)
        pltpu.sync_copy(rb, hA.at[pl.ds(r0, RPT)])
        pltpu.sync_copy(rb, h4.at[0, pl.ds(r0, RPT)])
        plsc.subcore_barrier()

        zero_sh(hB)
        plsc.subcore_barrier()
        prop(hA, hB)
        pltpu.sync_copy(hB.at[pl.ds(r0, RPT)], h4.at[1, pl.ds(r0, RPT)])
        zero_sh(hA)
        plsc.subcore_barrier()
        prop(hB, hA)
        pltpu.sync_copy(hA.at[pl.ds(r0, RPT)], h4.at[2, pl.ds(r0, RPT)])
        zero_sh(hB)
        plsc.subcore_barrier()
        prop(hA, hB)
        pltpu.sync_copy(hB.at[pl.ds(r0, RPT)], h4.at[3, pl.ds(r0, RPT)])


def _make_sc_kernel():
    mesh = plsc.VectorSubcoreMesh(core_axis_name="c", subcore_axis_name="s")
    scratch = [
        pltpu.VMEM((NCH, C), jnp.int32),
        pltpu.VMEM((NCH, C), jnp.int32),
        _scf32((NCH, C)),
        _scf32((NCH, C)),
        _scf32((NP,)),
        _scf32((C, H1)),
        _scf32((C,)),
        _scf32((C, H1)),
        _scf32((C, H1)),
        _scf32((C, H1)),
        _scf32((C, H1)),
        pltpu.SemaphoreType.DMA,
        pltpu.SemaphoreType.DMA,
        pltpu.SemaphoreType.DMA,
        _scf32((RPT, H1)),
        _scf32((C, H1)),
        _scf32((16,)),
        pltpu.VMEM_SHARED((NP, H1), jnp.float32),
        pltpu.VMEM_SHARED((NP,), jnp.float32),
        pltpu.VMEM_SHARED((NP, H1), jnp.float32),
        pltpu.VMEM_SHARED((NP, H1), jnp.float32),
    ]
    return pl.kernel(
        _sc_body,
        out_type=jax.ShapeDtypeStruct((4, NP, H1), jnp.float32),
        mesh=mesh,
        scratch_types=scratch,
        compiler_params=pltpu.CompilerParams(
            needs_layout_passes=False, use_tc_tiling_on_sc=False),
    )


def _tc_proj(x_ref, w_ref, o_ref):
    o_ref[:] = jnp.dot(x_ref[:], w_ref[:], preferred_element_type=jnp.float32)


def _tc_head(h4_ref, w2_ref, b2_ref, batch_ref, batchc_ref, wfc_ref, bfc_ref, o_ref):
    h4 = h4_ref[:]
    h2 = jnp.maximum(jnp.dot(h4, w2_ref[:], preferred_element_type=jnp.float32)
                     + b2_ref[:], 0.0)
    batch = batch_ref[:]
    gids = lax.broadcasted_iota(jnp.int32, (G, NP), 0)
    oh = (gids == batch).astype(jnp.float32)
    cnt = jnp.sum(oh, axis=1, keepdims=True)
    sums = jnp.dot(oh, h2, preferred_element_type=jnp.float32)
    mean = sums / jnp.clip(cnt, 1.0)
    neg = jnp.float32(-3.0e38)
    batchc = batchc_ref[:]
    mxs = []
    for g in range(G):
        maskg = batchc == g
        mxs.append(jnp.max(jnp.where(maskg, h2, neg), axis=0, keepdims=True))
    mx = jnp.concatenate(mxs, axis=0)
    mx = jnp.where(cnt > 0.0, mx, 0.0)
    pooled = jnp.concatenate([mx, mean], axis=1)
    o_ref[:] = jnp.dot(pooled, wfc_ref[:], preferred_element_type=jnp.float32) + bfc_ref[:]


@jax.jit
def kernel(x, edge_index, batch, edge_weight, W1, b1, W2, b2, Wfc, bfc):
    W1s = jnp.transpose(W1, (1, 0, 2)).reshape(F_IN, 32)
    W2s = W2.reshape(32, H2)

    row = edge_index[0]
    col = edge_index[1]
    pad_e = EP - E
    rowp = jnp.pad(row, (0, pad_e)).reshape(NTILES, NCH, C)
    colp = jnp.pad(col, (0, pad_e)).reshape(NTILES, NCH, C)
    ewp = jnp.pad(edge_weight, (0, pad_e)).reshape(NTILES, NCH, C)
    b1p = jnp.concatenate([b1, b1])

    y1 = pl.pallas_call(
        _tc_proj,
        out_shape=jax.ShapeDtypeStruct((N, 32), jnp.float32),
    )(x, W1s)
    y1p = jnp.pad(y1, ((0, NP - N), (0, 0)))
    y1p = jnp.transpose(y1p.reshape(NP, 4, H1), (1, 0, 2))

    h4 = _make_sc_kernel()(rowp, colp, ewp, y1p, b1p)
    h4 = jnp.transpose(h4, (1, 0, 2)).reshape(NP, 32)

    batchp = jnp.pad(batch, (0, NP - N), constant_values=G + 7)
    out = pl.pallas_call(
        _tc_head,
        out_shape=jax.ShapeDtypeStruct((G, 2), jnp.float32),
    )(h4, W2s, b2.reshape(1, H2), batchp.reshape(1, NP),
      batchp.reshape(NP, 1), Wfc, bfc.reshape(1, 2))
    return out


# speedup vs baseline: 25.4261x; 1.0030x over previous
import functools

import jax
import jax.numpy as jnp
from jax import lax
from jax.experimental import pallas as pl
from jax.experimental.pallas import tpu as pltpu
from jax.experimental.pallas import tpu_sc as plsc

N = 10000
E = 320000
F_IN = 128
H1 = 8
H2 = 16
G = 16

NTILES = 16
NP = 10240
RPT = NP // NTILES
C = 128
EPT = E // NTILES
NCH = (EPT + C - 1) // C
EPTP = NCH * C
EP = EPTP * NTILES


def _scf32(shape):
    return pltpu.VMEM(shape, jnp.float32)


def _sc_body(rowp, colp, ewp, y1p, b1p, h4,
             row2, col2, ew2, norm2, dis_l, comb, disc,
             gath, msg, gath2, msg2, gsem, gsem2, ssem, rb, zb, b1l,
             deg_sh, dis_sh, hA, hB):
    cid = lax.axis_index("c")
    wid = lax.axis_index("s")

    iot = lax.iota(jnp.int32, 16)
    hi = lax.shift_right_logical(iot, 3)
    lo = lax.bitwise_and(iot, 7)
    zf = jnp.zeros((16,), jnp.float32)

    @pl.when(cid == 0)
    def _main():
        r0 = wid * RPT

        pltpu.sync_copy(rowp.at[wid], row2)
        pltpu.sync_copy(colp.at[wid], col2)
        pltpu.sync_copy(ewp.at[wid], ew2)
        pltpu.sync_copy(b1p, b1l)

        for v in range(C * H1 // 16):
            plsc.store_scatter(zb, [hi + (2 * v), lo], zf)
            plsc.store_scatter(msg, [hi + (2 * v), lo], zf)
            plsc.store_scatter(msg2, [hi + (2 * v), lo], zf)
        for s in range(RPT // C):
            pltpu.sync_copy(zb, deg_sh.at[pl.ds(r0 + s * C, C)])
        plsc.subcore_barrier()

        zcol = jnp.zeros((16,), jnp.int32)
        def fill_ew(mbuf, j):
            for v in range(C // 16):
                w = ew2[j, pl.ds(v * 16, 16)]
                plsc.store_scatter(mbuf, [iot + 16 * v, zcol], w)

        def _deg(t, _):
            j0 = 2 * t
            j1 = j0 + 1
            fill_ew(msg, j0)
            d0 = pltpu.make_async_copy(msg, deg_sh.at[col2.at[j0]], ssem)
            d0.start(add=True)
            fill_ew(msg2, j1)
            d0.wait()
            pltpu.sync_copy(msg2, deg_sh.at[col2.at[j1]], add=True)
            return 0
        lax.fori_loop(0, (NCH - 1) // 2, _deg, 0)
        fill_ew(msg, NCH - 1)
        pltpu.sync_copy(msg, deg_sh.at[col2.at[NCH - 1]], add=True)
        plsc.subcore_barrier()

        for s in range(RPT // C):
            base = r0 + s * C
            pltpu.sync_copy(deg_sh.at[pl.ds(base, C)], comb)
            for v in range(C // 16):
                acc = plsc.load_gather(comb, [iot + 16 * v, zcol])
                m = acc > 0.0
                xs = jnp.where(m, acc, 1.0)
                i32 = plsc.bitcast(xs, jnp.int32)
                i32 = 1597463007 - lax.shift_right_arithmetic(i32, 1)
                y = plsc.bitcast(i32, jnp.float32)
                for _ in range(3):
                    y = y * (1.5 - 0.5 * xs * y * y)
                disc[pl.ds(v * 16, 16)] = jnp.where(m, y, 0.0)
            pltpu.sync_copy(disc, dis_sh.at[pl.ds(base, C)])
        plsc.subcore_barrier()
        pltpu.sync_copy(dis_sh, dis_l)

        def _norm(j, _):
            for v in range(C // 16):
                r = row2[j, pl.ds(v * 16, 16)]
                c = col2[j, pl.ds(v * 16, 16)]
                w = ew2[j, pl.ds(v * 16, 16)]
                n = plsc.load_gather(dis_l, [r]) * w * plsc.load_gather(dis_l, [c])
                norm2[j, pl.ds(v * 16, 16)] = n
            return 0
        lax.fori_loop(0, NCH, _norm, 0)
        plsc.subcore_barrier()

        def prop(src_sh, dst_sh):
            def scale(gbuf, mbuf, j):
                for w in range(C // 16):
                    nch = norm2[j, pl.ds(w * 16, 16)]
                    for u in range(8):
                        v = w * 8 + u
                        i0 = hi + (2 * v)
                        nrm = nch.at[hi + (2 * u)].get(mode="promise_in_bounds")
                        g = plsc.load_gather(gbuf, [i0, lo])
                        plsc.store_scatter(mbuf, [i0, lo], g * nrm)

            pltpu.async_copy(src_sh.at[row2.at[0]], gath, gsem)

            def _p(t, _):
                j0 = 2 * t
                j1 = j0 + 1
                pltpu.make_async_copy(src_sh.at[row2.at[j0]], gath, gsem).wait()
                pltpu.async_copy(src_sh.at[row2.at[j1]], gath2, gsem2)
                scale(gath, msg, j0)
                d0 = pltpu.make_async_copy(msg, dst_sh.at[col2.at[j0]], ssem)
                d0.start(add=True)
                pltpu.make_async_copy(src_sh.at[row2.at[j1]], gath2, gsem2).wait()

                @pl.when(t < (NCH - 1) // 2 - 1)
                def _pref():
                    pltpu.async_copy(src_sh.at[row2.at[j0 + 2]], gath, gsem)
                scale(gath2, msg2, j1)
                d0.wait()
                pltpu.sync_copy(msg2, dst_sh.at[col2.at[j1]], add=True)
                return 0
            lax.fori_loop(0, (NCH - 1) // 2, _p, 0)
            jl = NCH - 1
            pltpu.sync_copy(src_sh.at[row2.at[jl]], gath)
            scale(gath, msg, jl)
            pltpu.sync_copy(msg, dst_sh.at[col2.at[jl]], add=True)
            plsc.subcore_barrier()

        def init_cols(dst_sh, k):
            pltpu.sync_copy(y1p.at[k, pl.ds(r0, RPT)],
                            dst_sh.at[pl.ds(r0, RPT)])

        def zero_sh(dst_sh):
            for s in range(RPT // C):
                pltpu.sync_copy(zb, dst_sh.at[pl.ds(r0 + s * C, C)])

        init_cols(hA, 3)
        init_cols(hB, 2)
        plsc.subcore_barrier()
        prop(hA, hB)
        init_cols(hA, 1)
        plsc.subcore_barrier()
        prop(hB, hA)
        init_cols(hB, 0)
        plsc.subcore_barrier()
        prop(hA, hB)

        pltpu.sync_copy(hB.at[pl.ds(r0, RPT)], rb)
        bvec = b1l[pl.ds(0, 16)]
        def _relu(v, _):
            i0 = hi + (2 * v)
            val = plsc.load_gather(rb, [i0, lo])
            val = jnp.maximum(val + bvec, 0.0)
            plsc.store_scatter(rb, [i0, lo], val)
            return 0
        lax.fori_loop(0, RPT * H1 // 16, _relu, 0)
        pltpu.sync_copy(rb, hA.at[pl.ds(r0, RPT)])
        pltpu.sync_copy(rb, h4.at[0, pl.ds(r0, RPT)])
        plsc.subcore_barrier()

        zero_sh(hB)
        plsc.subcore_barrier()
        prop(hA, hB)
        pltpu.sync_copy(hB.at[pl.ds(r0, RPT)], h4.at[1, pl.ds(r0, RPT)])
        zero_sh(hA)
        plsc.subcore_barrier()
        prop(hB, hA)
        pltpu.sync_copy(hA.at[pl.ds(r0, RPT)], h4.at[2, pl.ds(r0, RPT)])
        zero_sh(hB)
        plsc.subcore_barrier()
        prop(hA, hB)
        pltpu.sync_copy(hB.at[pl.ds(r0, RPT)], h4.at[3, pl.ds(r0, RPT)])


def _make_sc_kernel():
    mesh = plsc.VectorSubcoreMesh(core_axis_name="c", subcore_axis_name="s")
    scratch = [
        pltpu.VMEM((NCH, C), jnp.int32),
        pltpu.VMEM((NCH, C), jnp.int32),
        _scf32((NCH, C)),
        _scf32((NCH, C)),
        _scf32((NP,)),
        _scf32((C, H1)),
        _scf32((C,)),
        _scf32((C, H1)),
        _scf32((C, H1)),
        _scf32((C, H1)),
        _scf32((C, H1)),
        pltpu.SemaphoreType.DMA,
        pltpu.SemaphoreType.DMA,
        pltpu.SemaphoreType.DMA,
        _scf32((RPT, H1)),
        _scf32((C, H1)),
        _scf32((16,)),
        pltpu.VMEM_SHARED((NP, H1), jnp.float32),
        pltpu.VMEM_SHARED((NP,), jnp.float32),
        pltpu.VMEM_SHARED((NP, H1), jnp.float32),
        pltpu.VMEM_SHARED((NP, H1), jnp.float32),
    ]
    return pl.kernel(
        _sc_body,
        out_type=jax.ShapeDtypeStruct((4, NP, H1), jnp.float32),
        mesh=mesh,
        scratch_types=scratch,
        compiler_params=pltpu.CompilerParams(
            needs_layout_passes=False, use_tc_tiling_on_sc=False),
    )


def _tc_proj(x_ref, w_ref, o_ref):
    o_ref[:] = jnp.dot(x_ref[:], w_ref[:], preferred_element_type=jnp.float32)


def _tc_head(h4_ref, w2_ref, b2_ref, batch_ref, batchc_ref, wfc_ref, bfc_ref, o_ref):
    h4 = h4_ref[:]
    h2 = jnp.maximum(jnp.dot(h4, w2_ref[:], preferred_element_type=jnp.float32)
                     + b2_ref[:], 0.0)
    batch = batch_ref[:]
    gids = lax.broadcasted_iota(jnp.int32, (G, NP), 0)
    oh = (gids == batch).astype(jnp.float32)
    cnt = jnp.sum(oh, axis=1, keepdims=True)
    sums = jnp.dot(oh, h2, preferred_element_type=jnp.float32)
    mean = sums / jnp.clip(cnt, 1.0)
    neg = jnp.float32(-3.0e38)
    batchc = batchc_ref[:]
    mxs = []
    for g in range(G):
        maskg = batchc == g
        mxs.append(jnp.max(jnp.where(maskg, h2, neg), axis=0, keepdims=True))
    mx = jnp.concatenate(mxs, axis=0)
    mx = jnp.where(cnt > 0.0, mx, 0.0)
    pooled = jnp.concatenate([mx, mean], axis=1)
    o_ref[:] = jnp.dot(pooled, wfc_ref[:], preferred_element_type=jnp.float32) + bfc_ref[:]


@jax.jit
def kernel(x, edge_index, batch, edge_weight, W1, b1, W2, b2, Wfc, bfc):
    W1s = jnp.transpose(W1, (1, 0, 2)).reshape(F_IN, 32)
    W2s = W2.reshape(32, H2)

    row = edge_index[0]
    col = edge_index[1]
    pad_e = EP - E
    rowp = jnp.pad(row, (0, pad_e)).reshape(NTILES, NCH, C)
    colp = jnp.pad(col, (0, pad_e)).reshape(NTILES, NCH, C)
    ewp = jnp.pad(edge_weight, (0, pad_e)).reshape(NTILES, NCH, C)
    b1p = jnp.concatenate([b1, b1])

    y1 = pl.pallas_call(
        _tc_proj,
        out_shape=jax.ShapeDtypeStruct((N, 32), jnp.float32),
    )(x, W1s)
    y1p = jnp.pad(y1, ((0, NP - N), (0, 0)))
    y1p = jnp.transpose(y1p.reshape(NP, 4, H1), (1, 0, 2))

    h4 = _make_sc_kernel()(rowp, colp, ewp, y1p, b1p)
    h4 = jnp.transpose(h4, (1, 0, 2)).reshape(NP, 32)

    batchp = jnp.pad(batch, (0, NP - N), constant_values=G + 7)
    out = pl.pallas_call(
        _tc_head,
        out_shape=jax.ShapeDtypeStruct((G, 2), jnp.float32),
    )(h4, W2s, b2.reshape(1, H2), batchp.reshape(1, NP),
      batchp.reshape(NP, 1), Wfc, bfc.reshape(1, 2))
    return out
